# baseline (device time: 24313 ns/iter reference)
import jax
import jax.numpy as jnp
from jax import lax
from jax.experimental import pallas as pl
from jax.experimental.pallas import tpu as pltpu

N_DEV = 8
M = 512
CHUNK = 512

SCHEDULES = [
    ("A0", ("x", "y", "z"), 0, 112),
    ("B0", ("y", "z", "x"), 176, 112),
    ("C0", ("z", "x", "y"), 352, 96),
    ("A1", ("x", "y", "z"), 112, 64),
    ("B1", ("y", "z", "x"), 288, 64),
    ("C1", ("z", "x", "y"), 448, 64),
]
NS = len(SCHEDULES)
XOR = {"x": 1, "y": 3, "z": 4}


def _snake_pos(cx, cy, cz):
    return cz * 4 + [[0, 3], [1, 2]][cx][cy]


def kernel(x):
    def body(x_ref, out_ref, *scr):
        bufs = [scr[4 * i] for i in range(NS)]
        recvs = [scr[4 * i + 1 : 4 * i + 4] for i in range(NS)]
        send_sems, recv_sems = scr[4 * NS], scr[4 * NS + 1]

        my = lax.axis_index("i")
        cx = (my & 1) ^ ((my >> 1) & 1)
        cy = (my >> 1) & 1
        cz = (my >> 2) & 1
        coord = {"x": cx, "y": cy, "z": cz}

        barrier = pltpu.get_barrier_semaphore()
        for ax in ("x", "y", "z"):
            pl.semaphore_signal(
                barrier,
                inc=1,
                device_id=(my ^ XOR[ax],),
                device_id_type=pl.DeviceIdType.MESH,
            )
        pl.semaphore_wait(barrier, 3)

        rdmas = [[None] * 3 for _ in range(NS)]
        kept = [None] * NS

        def pack_slots(si, slots):
            _, order, r0, rows = SCHEDULES[si]
            for s in slots:
                b = (s >> 2 & 1, s >> 1 & 1, s & 1)
                c = dict(zip(order, b))
                p = _snake_pos(c["x"], c["y"], c["z"])
                bufs[si][s] = x_ref[
                    0, r0 : r0 + rows, p * CHUNK : (p + 1) * CHUNK
                ].astype(jnp.bfloat16)

        def start_step(si, k):
            _, order, _, _ = SCHEDULES[si]
            c = coord[order[k]]
            size = 4 >> k
            base, _ = kept[si] if k else (0, 8)
            kept[si] = (base + c * size, size)
            send_base = base + (1 - c) * size
            rdma = pltpu.make_async_remote_copy(
                src_ref=bufs[si].at[pl.ds(send_base, size)],
                dst_ref=recvs[si][k],
                send_sem=send_sems.at[si, k],
                recv_sem=recv_sems.at[si, k],
                device_id=(my ^ XOR[order[k]],),
                device_id_type=pl.DeviceIdType.MESH,
            )
            rdma.start()
            rdmas[si][k] = rdma

        def add_sub(si, k, base, off, n):
            bufs[si][pl.ds(base + off, n)] = (
                bufs[si][pl.ds(base + off, n)] + recvs[si][k][pl.ds(off, n)]
            )

        for si in range(NS):
            c1 = coord[SCHEDULES[si][1][0]]
            pl.when(c1 == 0)(lambda si=si: pack_slots(si, range(4, 8)))
            pl.when(c1 == 1)(lambda si=si: pack_slots(si, range(0, 4)))
            start_step(si, 0)
        for si in range(NS):
            c1 = coord[SCHEDULES[si][1][0]]
            pl.when(c1 == 0)(lambda si=si: pack_slots(si, range(0, 4)))
            pl.when(c1 == 1)(lambda si=si: pack_slots(si, range(4, 8)))
        for k in range(2):
            for si in range(NS):
                rdmas[si][k].wait()
                base, size = kept[si]
                h = size // 2
                cn = coord[SCHEDULES[si][1][k + 1]]
                add_sub(si, k, base, (1 - cn) * h, h)
                start_step(si, k + 1)
                add_sub(si, k, base, cn * h, h)
        for si in range(NS):
            rdmas[si][2].wait()
            base, _ = kept[si]
            _, _, r0, rows = SCHEDULES[si]
            out_ref[r0 : r0 + rows, :] = bufs[si][base] + recvs[si][2][0]

    scratch_shapes = []
    for _, _, _, rows in SCHEDULES:
        scratch_shapes.append(pltpu.VMEM((8, rows, CHUNK), jnp.bfloat16))
        for k in range(3):
            scratch_shapes.append(
                pltpu.VMEM((4 >> k, rows, CHUNK), jnp.bfloat16)
            )
    scratch_shapes.append(pltpu.SemaphoreType.DMA((NS, 3)))
    scratch_shapes.append(pltpu.SemaphoreType.DMA((NS, 3)))

    return pl.pallas_call(
        body,
        out_shape=jax.ShapeDtypeStruct((M, CHUNK), jnp.bfloat16),
        in_specs=[pl.BlockSpec(memory_space=pltpu.VMEM)],
        out_specs=pl.BlockSpec(memory_space=pltpu.VMEM),
        scratch_shapes=scratch_shapes,
        compiler_params=pltpu.CompilerParams(collective_id=0),
    )(x)
